# baseline (speedup 1.0000x reference)
import jax
import jax.numpy as jnp
from jax import lax
from jax.experimental import pallas as pl
from jax.experimental.pallas import tpu as pltpu
from jax.experimental.pallas import tpu_sc as plsc

B, L, D = 16, 2048, 1024
DHS = D // 2
NV = DHS // 16
R = 64
GROUP = 16
NSUB = 16
BL = 512
F_NUM, F_DEN = 7, 8
NB = (F_NUM * L) // (F_DEN * BL)


def _ntc_blocks(n):
    return (F_NUM * n) // (F_DEN * BL)


def _row_add(buf, r, accs):
    return tuple(accs[v] + buf[r, pl.ds(v * 16, 16)] for v in range(NV))


def _acc_chunk(buf, accs):
    def group_body(g, accs):
        row0 = g * GROUP
        return lax.fori_loop(
            row0, row0 + GROUP, lambda r, a: _row_add(buf, r, a), accs,
            unroll=True,
        )

    return lax.fori_loop(0, R // GROUP, group_body, accs)


def _sc_body(feat_hbm, len_hbm, out_hbm, len_v, buf0, buf1, acc_v, tmp16,
             shared, sem0, sem1):
    core = lax.axis_index("c")
    t = lax.axis_index("s")
    col0 = core * DHS

    pltpu.sync_copy(len_hbm, len_v.at[pl.ds(0, 16)])

    def ln(b):
        return len_v[pl.ds(b, 16)][0]

    def sc_lo(b):
        return _ntc_blocks(ln(b)) * BL

    def ln_sc(b):
        return ln(b) - sc_lo(b)

    def nc(b):
        return (ln_sc(b) + (R - 1)) // R

    NC = lax.fori_loop(0, B, lambda i, c: c + nc(i), 0)
    Q = (NC + (NSUB - 1)) // NSUB
    g0 = jnp.minimum(t * Q, NC)
    g1 = jnp.minimum(g0 + Q, NC)

    def accumulate_span(b, r_lo, n):
        nfull = n // R
        npairs = nfull // 2
        odd = nfull - 2 * npairs
        tail = n - nfull * R

        def start(c, buf, sem):
            pltpu.async_copy(
                feat_hbm.at[b, pl.ds(r_lo + c * R, R), pl.ds(col0, DHS)],
                buf, sem,
            )

        def wait(buf, sem):
            pltpu.make_async_copy(
                feat_hbm.at[0, pl.ds(0, R), pl.ds(0, DHS)], buf, sem
            ).wait()

        @pl.when(nfull >= 1)
        def _():
            start(0, buf0, sem0)

        @pl.when(nfull >= 2)
        def _():
            start(1, buf1, sem1)

        def pair_body(i, accs):
            wait(buf0, sem0)
            accs = _acc_chunk(buf0, accs)

            @pl.when(2 * i + 2 < nfull)
            def _():
                start(2 * i + 2, buf0, sem0)

            wait(buf1, sem1)
            accs = _acc_chunk(buf1, accs)

            @pl.when(2 * i + 3 < nfull)
            def _():
                start(2 * i + 3, buf1, sem1)

            return accs

        accs0 = tuple(jnp.zeros((16,), jnp.float32) for _ in range(NV))
        accs = lax.fori_loop(0, npairs, pair_body, accs0)

        t0 = jnp.minimum(r_lo + nfull * R, L - R)
        off = r_lo + nfull * R - t0

        @pl.when(tail > 0)
        def _():
            pltpu.async_copy(
                feat_hbm.at[b, pl.ds(t0, R), pl.ds(col0, DHS)], buf1, sem1
            )

        @pl.when(odd > 0)
        def _():
            wait(buf0, sem0)

        accs = lax.fori_loop(0, odd * R, lambda r, a: _row_add(buf0, r, a),
                             accs)

        @pl.when(tail > 0)
        def _():
            wait(buf1, sem1)

        accs = lax.fori_loop(off, off + tail,
                             lambda r, a: _row_add(buf1, r, a), accs)
        return accs

    def seek(th):
        def step(i, st):
            b, cum = st
            ncb = nc(b)
            adv = (b < B) & (cum + ncb <= th)
            return (
                jnp.where(adv, b + 1, b),
                jnp.where(adv, cum + ncb, cum),
            )

        return lax.fori_loop(0, B, step, (jnp.int32(0), jnp.int32(0)))

    b0, cum0 = seek(g0)
    b_end, _ = seek(g1 - 1)
    nbat = jnp.where(g1 > g0, b_end - b0 + 1, 0)

    def walk_body(i, st):
        b, cum = st
        ncb = nc(b)
        j_lo = jnp.maximum(g0 - cum, 0)
        j_hi = jnp.minimum(g1 - cum, ncb)
        base = sc_lo(b)
        r_lo = base + j_lo * R
        r_hi = jnp.minimum(base + j_hi * R, ln(b))
        accs = accumulate_span(b, r_lo, r_hi - r_lo)
        for v in range(NV):
            acc_v[pl.ds(v * 16, 16)] = accs[v]
        pltpu.sync_copy(acc_v, shared.at[t, b])
        return (b + 1, cum + ncb)

    lax.fori_loop(0, nbat, walk_body, (b0, cum0))

    plsc.subcore_barrier()

    cum_t = lax.fori_loop(0, t, lambda i, c: c + nc(i), 0)
    nct = nc(t)
    lt = ln(t)
    pltpu.sync_copy(shared.at[:, t], tmp16)

    accs = tuple(jnp.zeros((16,), jnp.float32) for _ in range(NV))
    for tp in range(NSUB):
        touched = (tp * Q < cum_t + nct) & (tp * Q + Q > cum_t)
        m = jnp.broadcast_to(touched.astype(jnp.float32), (16,))
        accs = tuple(
            accs[v] + tmp16[tp, pl.ds(v * 16, 16)] * m for v in range(NV)
        )

    lenf = jnp.broadcast_to(lt.astype(jnp.float32), (16,))
    inv = jnp.ones((16,), jnp.float32) / lenf
    for v in range(NV):
        acc_v[pl.ds(v * 16, 16)] = accs[v] * inv
    pltpu.sync_copy(acc_v, out_hbm.at[t, pl.ds(core * DHS, DHS)])


def _sc_pool(features, lengths32):
    mesh = plsc.VectorSubcoreMesh(core_axis_name="c", subcore_axis_name="s")
    f = pl.kernel(
        _sc_body,
        out_type=jax.ShapeDtypeStruct((B, D), jnp.float32),
        mesh=mesh,
        scratch_types=[
            pltpu.VMEM((32,), jnp.int32),
            pltpu.VMEM((R, DHS), jnp.float32),
            pltpu.VMEM((R, DHS), jnp.float32),
            pltpu.VMEM((DHS,), jnp.float32),
            pltpu.VMEM((NSUB, DHS), jnp.float32),
            pltpu.VMEM_SHARED((NSUB, B, DHS), jnp.float32),
            pltpu.SemaphoreType.DMA,
            pltpu.SemaphoreType.DMA,
        ],
    )
    return f(features, lengths32)


def _tc_kernel(lens_ref, feat_ref, out_ref, acc_scr):
    i = pl.program_id(0)
    j = pl.program_id(1)
    ln = lens_ref[i]
    ntc = _ntc_blocks(ln)

    @pl.when(j == 0)
    def _():
        acc_scr[...] = jnp.zeros_like(acc_scr)

    @pl.when(j < ntc)
    def _():
        acc = feat_ref[0, 0:8, :]
        for k in range(1, BL // 8):
            acc = acc + feat_ref[0, k * 8:(k + 1) * 8, :]
        acc_scr[...] += acc

    @pl.when(j == NB - 1)
    def _():
        s = jnp.sum(acc_scr[...], axis=0, keepdims=True)
        out_ref[...] = (s / ln.astype(jnp.float32))[None]


def _tc_pool(features, lengths32):
    grid_spec = pltpu.PrefetchScalarGridSpec(
        num_scalar_prefetch=1,
        grid=(B, NB),
        in_specs=[
            pl.BlockSpec(
                (1, BL, D),
                lambda i, j, lens: (
                    i,
                    jnp.maximum(
                        jnp.minimum(j, _ntc_blocks(lens[i]) - 1), 0
                    ),
                    0,
                ),
            ),
        ],
        out_specs=pl.BlockSpec((1, 1, D), lambda i, j, lens: (i, 0, 0)),
        scratch_shapes=[pltpu.VMEM((8, D), jnp.float32)],
    )
    out = pl.pallas_call(
        _tc_kernel,
        grid_spec=grid_spec,
        out_shape=jax.ShapeDtypeStruct((B, 1, D), jnp.float32),
        compiler_params=pltpu.CompilerParams(
            dimension_semantics=("arbitrary", "arbitrary"),
        ),
    )(lengths32, features)
    return out[:, 0, :]


def kernel(features, lengths):
    lengths32 = lengths.astype(jnp.int32)
    out_tc = _tc_pool(features, lengths32)
    out_sc = _sc_pool(features, lengths32)
    return out_tc + out_sc


# speedup vs baseline: 1.0975x; 1.0975x over previous
import jax
import jax.numpy as jnp
from jax import lax
from jax.experimental import pallas as pl
from jax.experimental.pallas import tpu as pltpu
from jax.experimental.pallas import tpu_sc as plsc

B, L, D = 16, 2048, 1024
DHS = D // 2
NV = DHS // 16
R = 64
GROUP = 16
NSUB = 16
BL = 512
F_NUM, F_DEN = 3, 4
NB = (F_NUM * L) // (F_DEN * BL)


def _ntc_blocks(n):
    return (F_NUM * n) // (F_DEN * BL)


def _row_add(buf, r, accs):
    return tuple(accs[v] + buf[r, pl.ds(v * 16, 16)] for v in range(NV))


def _acc_chunk(buf, accs):
    def group_body(g, accs):
        row0 = g * GROUP
        return lax.fori_loop(
            row0, row0 + GROUP, lambda r, a: _row_add(buf, r, a), accs,
            unroll=True,
        )

    return lax.fori_loop(0, R // GROUP, group_body, accs)


def _sc_body(feat_hbm, len_hbm, out_hbm, len_v, buf0, buf1, acc_v, tmp16,
             shared, sem0, sem1):
    core = lax.axis_index("c")
    t = lax.axis_index("s")
    col0 = core * DHS

    pltpu.sync_copy(len_hbm, len_v.at[pl.ds(0, 16)])

    def ln(b):
        return len_v[pl.ds(b, 16)][0]

    def sc_lo(b):
        return _ntc_blocks(ln(b)) * BL

    def ln_sc(b):
        return ln(b) - sc_lo(b)

    def nc(b):
        return (ln_sc(b) + (R - 1)) // R

    NC = lax.fori_loop(0, B, lambda i, c: c + nc(i), 0)
    Q = (NC + (NSUB - 1)) // NSUB
    g0 = jnp.minimum(t * Q, NC)
    g1 = jnp.minimum(g0 + Q, NC)

    def accumulate_span(b, r_lo, n):
        nfull = n // R
        npairs = nfull // 2
        odd = nfull - 2 * npairs
        tail = n - nfull * R

        def start(c, buf, sem):
            pltpu.async_copy(
                feat_hbm.at[b, pl.ds(r_lo + c * R, R), pl.ds(col0, DHS)],
                buf, sem,
            )

        def wait(buf, sem):
            pltpu.make_async_copy(
                feat_hbm.at[0, pl.ds(0, R), pl.ds(0, DHS)], buf, sem
            ).wait()

        @pl.when(nfull >= 1)
        def _():
            start(0, buf0, sem0)

        @pl.when(nfull >= 2)
        def _():
            start(1, buf1, sem1)

        def pair_body(i, accs):
            wait(buf0, sem0)
            accs = _acc_chunk(buf0, accs)

            @pl.when(2 * i + 2 < nfull)
            def _():
                start(2 * i + 2, buf0, sem0)

            wait(buf1, sem1)
            accs = _acc_chunk(buf1, accs)

            @pl.when(2 * i + 3 < nfull)
            def _():
                start(2 * i + 3, buf1, sem1)

            return accs

        accs0 = tuple(jnp.zeros((16,), jnp.float32) for _ in range(NV))
        accs = lax.fori_loop(0, npairs, pair_body, accs0)

        t0 = jnp.minimum(r_lo + nfull * R, L - R)
        off = r_lo + nfull * R - t0

        @pl.when(tail > 0)
        def _():
            pltpu.async_copy(
                feat_hbm.at[b, pl.ds(t0, R), pl.ds(col0, DHS)], buf1, sem1
            )

        @pl.when(odd > 0)
        def _():
            wait(buf0, sem0)

        accs = lax.fori_loop(0, odd * R, lambda r, a: _row_add(buf0, r, a),
                             accs)

        @pl.when(tail > 0)
        def _():
            wait(buf1, sem1)

        accs = lax.fori_loop(off, off + tail,
                             lambda r, a: _row_add(buf1, r, a), accs)
        return accs

    def seek(th):
        def step(i, st):
            b, cum = st
            ncb = nc(b)
            adv = (b < B) & (cum + ncb <= th)
            return (
                jnp.where(adv, b + 1, b),
                jnp.where(adv, cum + ncb, cum),
            )

        return lax.fori_loop(0, B, step, (jnp.int32(0), jnp.int32(0)))

    b0, cum0 = seek(g0)
    b_end, _ = seek(g1 - 1)
    nbat = jnp.where(g1 > g0, b_end - b0 + 1, 0)

    def walk_body(i, st):
        b, cum = st
        ncb = nc(b)
        j_lo = jnp.maximum(g0 - cum, 0)
        j_hi = jnp.minimum(g1 - cum, ncb)
        base = sc_lo(b)
        r_lo = base + j_lo * R
        r_hi = jnp.minimum(base + j_hi * R, ln(b))
        accs = accumulate_span(b, r_lo, r_hi - r_lo)
        for v in range(NV):
            acc_v[pl.ds(v * 16, 16)] = accs[v]
        pltpu.sync_copy(acc_v, shared.at[t, b])
        return (b + 1, cum + ncb)

    lax.fori_loop(0, nbat, walk_body, (b0, cum0))

    plsc.subcore_barrier()

    cum_t = lax.fori_loop(0, t, lambda i, c: c + nc(i), 0)
    nct = nc(t)
    lt = ln(t)
    pltpu.sync_copy(shared.at[:, t], tmp16)

    accs = tuple(jnp.zeros((16,), jnp.float32) for _ in range(NV))
    for tp in range(NSUB):
        touched = (tp * Q < cum_t + nct) & (tp * Q + Q > cum_t)
        m = jnp.broadcast_to(touched.astype(jnp.float32), (16,))
        accs = tuple(
            accs[v] + tmp16[tp, pl.ds(v * 16, 16)] * m for v in range(NV)
        )

    lenf = jnp.broadcast_to(lt.astype(jnp.float32), (16,))
    inv = jnp.ones((16,), jnp.float32) / lenf
    for v in range(NV):
        acc_v[pl.ds(v * 16, 16)] = accs[v] * inv
    pltpu.sync_copy(acc_v, out_hbm.at[t, pl.ds(core * DHS, DHS)])


def _sc_pool(features, lengths32):
    mesh = plsc.VectorSubcoreMesh(core_axis_name="c", subcore_axis_name="s")
    f = pl.kernel(
        _sc_body,
        out_type=jax.ShapeDtypeStruct((B, D), jnp.float32),
        mesh=mesh,
        scratch_types=[
            pltpu.VMEM((32,), jnp.int32),
            pltpu.VMEM((R, DHS), jnp.float32),
            pltpu.VMEM((R, DHS), jnp.float32),
            pltpu.VMEM((DHS,), jnp.float32),
            pltpu.VMEM((NSUB, DHS), jnp.float32),
            pltpu.VMEM_SHARED((NSUB, B, DHS), jnp.float32),
            pltpu.SemaphoreType.DMA,
            pltpu.SemaphoreType.DMA,
        ],
    )
    return f(features, lengths32)


def _tc_kernel(lens_ref, feat_ref, out_ref, acc_scr):
    i = pl.program_id(0)
    j = pl.program_id(1)
    ln = lens_ref[i]
    ntc = _ntc_blocks(ln)

    @pl.when(j == 0)
    def _():
        acc_scr[...] = jnp.zeros_like(acc_scr)

    @pl.when(j < ntc)
    def _():
        acc = feat_ref[0, 0:8, :]
        for k in range(1, BL // 8):
            acc = acc + feat_ref[0, k * 8:(k + 1) * 8, :]
        acc_scr[...] += acc

    @pl.when(j == NB - 1)
    def _():
        s = jnp.sum(acc_scr[...], axis=0, keepdims=True)
        out_ref[...] = (s / ln.astype(jnp.float32))[None]


def _tc_pool(features, lengths32):
    grid_spec = pltpu.PrefetchScalarGridSpec(
        num_scalar_prefetch=1,
        grid=(B, NB),
        in_specs=[
            pl.BlockSpec(
                (1, BL, D),
                lambda i, j, lens: (
                    i,
                    jnp.maximum(
                        jnp.minimum(j, _ntc_blocks(lens[i]) - 1), 0
                    ),
                    0,
                ),
            ),
        ],
        out_specs=pl.BlockSpec((1, 1, D), lambda i, j, lens: (i, 0, 0)),
        scratch_shapes=[pltpu.VMEM((8, D), jnp.float32)],
    )
    out = pl.pallas_call(
        _tc_kernel,
        grid_spec=grid_spec,
        out_shape=jax.ShapeDtypeStruct((B, 1, D), jnp.float32),
        compiler_params=pltpu.CompilerParams(
            dimension_semantics=("arbitrary", "arbitrary"),
        ),
    )(lengths32, features)
    return out[:, 0, :]


def kernel(features, lengths):
    lengths32 = lengths.astype(jnp.int32)
    out_tc = _tc_pool(features, lengths32)
    out_sc = _sc_pool(features, lengths32)
    return out_tc + out_sc
